# baseline (speedup 1.0000x reference)
import functools

import jax
import jax.numpy as jnp
from jax import lax
from jax.experimental import pallas as pl
from jax.experimental.pallas import tpu as pltpu
from jax.experimental.pallas import tpu_sc as plsc

_C = 21
_TBL = 448


def _argmax_idx_body(x_ref, y_ref, o_ref):
    m = x_ref[0, 0]
    for c in range(1, _C):
        m = jnp.maximum(m, x_ref[0, c])
    a = jnp.where(x_ref[0, 0] == m, 0.0, float(_C))
    for c in range(1, _C):
        a = jnp.minimum(a, jnp.where(x_ref[0, c] == m, float(c), float(_C)))
    idx = y_ref[0] * _C + a.astype(jnp.int32)
    half = idx.shape[0] // 2
    o_ref[0] = idx[:half] | (idx[half:] << 16)


def _cost_index(inp, y):
    b, c, h, w = inp.shape
    hb = 128
    return pl.pallas_call(
        _argmax_idx_body,
        grid=(b, h // hb),
        in_specs=[
            pl.BlockSpec((1, c, hb, w), lambda i, j: (i, 0, j, 0)),
            pl.BlockSpec((1, hb, w), lambda i, j: (i, j, 0)),
        ],
        out_specs=pl.BlockSpec((1, hb // 2, w), lambda i, j: (i, j, 0)),
        out_shape=jax.ShapeDtypeStruct((b, h // 2, w), jnp.int32),
    )(inp, y)


def _make_sc_reduce(n):
    info = plsc.get_sparse_core_info()
    nc, ns, lanes = info.num_cores, info.num_subcores, info.num_lanes
    nw = nc * ns
    per_w = n // nw
    mesh = plsc.VectorSubcoreMesh(core_axis_name="c", subcore_axis_name="s")

    per_w32 = per_w // 2
    nch = 4
    ch = per_w32 // nch

    @functools.partial(
        pl.kernel,
        mesh=mesh,
        compiler_params=pltpu.CompilerParams(needs_layout_passes=False),
        out_type=jax.ShapeDtypeStruct((nw * lanes,), jnp.float32),
        scratch_types=[
            pltpu.VMEM((2, ch), jnp.int32),
            pltpu.VMEM((_C * _C,), jnp.float32),
            pltpu.VMEM((lanes,), jnp.float32),
            pltpu.SemaphoreType.DMA,
            pltpu.SemaphoreType.DMA,
        ],
    )
    def sc_reduce(idx_hbm, tbl_hbm, out_hbm, idx_v, tbl_v, acc_v, s0, s1):
        wid = lax.axis_index("s") * nc + lax.axis_index("c")
        base = wid * per_w32
        sems = (s0, s1)
        handles = [None, None]
        handles[0] = pltpu.async_copy(
            idx_hbm.at[pl.ds(base, ch)], idx_v.at[0], sems[0])
        pltpu.sync_copy(tbl_hbm, tbl_v)

        unroll = 2

        def chunk_body(buf):
            def body(j, accs):
                jb = j * (unroll * lanes)
                out = []
                for u in range(unroll):
                    packed = idx_v[buf, pl.ds(jb + u * lanes, lanes)]
                    lo = packed & 0xFFFF
                    hi = lax.shift_right_logical(packed, 16)
                    out.append(accs[2 * u] + plsc.load_gather(tbl_v, [lo]))
                    out.append(accs[2 * u + 1] + plsc.load_gather(tbl_v, [hi]))
                return tuple(out)
            return body

        z = jnp.zeros((lanes,), jnp.float32)
        accs = (z,) * (2 * unroll)
        for k in range(nch):
            if k + 1 < nch:
                handles[(k + 1) % 2] = pltpu.async_copy(
                    idx_hbm.at[pl.ds(base + (k + 1) * ch, ch)],
                    idx_v.at[(k + 1) % 2], sems[(k + 1) % 2])
            handles[k % 2].wait()
            accs = lax.fori_loop(0, ch // (unroll * lanes),
                                 chunk_body(k % 2), accs)
        acc_v[...] = (accs[0] + accs[1]) + (accs[2] + accs[3])
        pltpu.sync_copy(acc_v, out_hbm.at[pl.ds(wid * lanes, lanes)])

    return sc_reduce


def kernel(input, y_true, D):
    b, c, h, w = input.shape
    n = b * h * w
    idx32 = _cost_index(input, y_true).reshape(n // 2)
    partials = _make_sc_reduce(n)(idx32, D.reshape(c * c))
    return jnp.sum(partials) / n


# speedup vs baseline: 1.0064x; 1.0064x over previous
import functools

import jax
import jax.numpy as jnp
from jax import lax
from jax.experimental import pallas as pl
from jax.experimental.pallas import tpu as pltpu
from jax.experimental.pallas import tpu_sc as plsc

_C = 21


def _argmax_idx_body(x_ref, y_ref, o_ref):
    m = x_ref[0, 0]
    for c in range(1, _C):
        m = jnp.maximum(m, x_ref[0, c])
    a = jnp.where(x_ref[0, 0] == m, 0.0, float(_C))
    for c in range(1, _C):
        a = jnp.minimum(a, jnp.where(x_ref[0, c] == m, float(c), float(_C)))
    idx = y_ref[0] * _C + a.astype(jnp.int32)
    half = idx.shape[0] // 2
    o_ref[0] = idx[:half] | (idx[half:] << 16)


def _cost_index(inp, y):
    b, c, h, w = inp.shape
    hb = 256
    return pl.pallas_call(
        _argmax_idx_body,
        grid=(b, h // hb),
        in_specs=[
            pl.BlockSpec((1, c, hb, w), lambda i, j: (i, 0, j, 0)),
            pl.BlockSpec((1, hb, w), lambda i, j: (i, j, 0)),
        ],
        out_specs=pl.BlockSpec((1, hb // 2, w), lambda i, j: (i, j, 0)),
        out_shape=jax.ShapeDtypeStruct((b, h // 2, w), jnp.int32),
    )(inp, y)


def _make_sc_reduce(n):
    info = plsc.get_sparse_core_info()
    nc, ns, lanes = info.num_cores, info.num_subcores, info.num_lanes
    nw = nc * ns
    per_w = n // nw
    mesh = plsc.VectorSubcoreMesh(core_axis_name="c", subcore_axis_name="s")

    per_w32 = per_w // 2
    nch = 4
    ch = per_w32 // nch

    @functools.partial(
        pl.kernel,
        mesh=mesh,
        compiler_params=pltpu.CompilerParams(needs_layout_passes=False),
        out_type=jax.ShapeDtypeStruct((nw * lanes,), jnp.float32),
        scratch_types=[
            pltpu.VMEM((2, ch), jnp.int32),
            pltpu.VMEM((_C * _C,), jnp.float32),
            pltpu.VMEM((lanes,), jnp.float32),
            pltpu.SemaphoreType.DMA,
            pltpu.SemaphoreType.DMA,
        ],
    )
    def sc_reduce(idx_hbm, tbl_hbm, out_hbm, idx_v, tbl_v, acc_v, s0, s1):
        wid = lax.axis_index("s") * nc + lax.axis_index("c")
        base = wid * per_w32
        sems = (s0, s1)
        handles = [None, None]
        handles[0] = pltpu.async_copy(
            idx_hbm.at[pl.ds(base, ch)], idx_v.at[0], sems[0])
        pltpu.sync_copy(tbl_hbm, tbl_v)

        unroll = 2

        def chunk_body(buf):
            def body(j, accs):
                jb = j * (unroll * lanes)
                out = []
                for u in range(unroll):
                    packed = idx_v[buf, pl.ds(jb + u * lanes, lanes)]
                    lo = packed & 0xFFFF
                    hi = lax.shift_right_logical(packed, 16)
                    out.append(accs[2 * u] + plsc.load_gather(tbl_v, [lo]))
                    out.append(accs[2 * u + 1] + plsc.load_gather(tbl_v, [hi]))
                return tuple(out)
            return body

        z = jnp.zeros((lanes,), jnp.float32)
        accs = (z,) * (2 * unroll)
        for k in range(nch):
            if k + 1 < nch:
                handles[(k + 1) % 2] = pltpu.async_copy(
                    idx_hbm.at[pl.ds(base + (k + 1) * ch, ch)],
                    idx_v.at[(k + 1) % 2], sems[(k + 1) % 2])
            handles[k % 2].wait()
            accs = lax.fori_loop(0, ch // (unroll * lanes),
                                 chunk_body(k % 2), accs)
        acc_v[...] = (accs[0] + accs[1]) + (accs[2] + accs[3])
        pltpu.sync_copy(acc_v, out_hbm.at[pl.ds(wid * lanes, lanes)])

    return sc_reduce


def kernel(input, y_true, D):
    b, c, h, w = input.shape
    n = b * h * w
    idx32 = _cost_index(input, y_true).reshape(n // 2)
    partials = _make_sc_reduce(n)(idx32, D.reshape(c * c))
    return jnp.sum(partials) / n
